# baseline (speedup 1.0000x reference)
import jax
import jax.numpy as jnp
from jax.experimental import pallas as pl
from jax.experimental.pallas import tpu as pltpu


def _lstm_seq_kernel(x_ref,
                     wi_ref,
                     wh_ref,
                     b_ref,
                     c0_ref,
                     h0_ref,
                     cy_ref,
                     hy_ref):
    T = x_ref.shape[0]
    B = c0_ref.shape[0]
    H = c0_ref.shape[1]
    Bh = B // 2

    def half(x_t, c, h):
        gates = (jnp.dot(x_t.astype(jnp.bfloat16), wi_ref[...],
                         preferred_element_type=jnp.float32)
                 + jnp.dot(h.astype(jnp.bfloat16), wh_ref[...],
                           preferred_element_type=jnp.float32)
                 + b_ref[...])
        ingate     = jax.nn.sigmoid(gates[:, 0 * H:1 * H])
        forgetgate = jax.nn.sigmoid(gates[:, 1 * H:2 * H])
        cellgate   = jnp.tanh(gates[:, 2 * H:3 * H])
        outgate    = jax.nn.sigmoid(gates[:, 3 * H:4 * H])
        cy = forgetgate * c + ingate * cellgate
        hy = outgate * jnp.tanh(cy)
        return cy, hy

    def step(t, carry):
        c0, h0, c1, h1 = carry
        x_t = x_ref[t]
        cy0, hy0 = half(x_t[:Bh], c0, h0)
        cy1, hy1 = half(x_t[Bh:], c1, h1)
        cy_ref[t, :Bh] = cy0
        hy_ref[t, :Bh] = hy0
        cy_ref[t, Bh:] = cy1
        hy_ref[t, Bh:] = hy1
        return (cy0, hy0, cy1, hy1)

    jax.lax.fori_loop(
        0, T, step,
        (c0_ref[:Bh], h0_ref[:Bh], c0_ref[Bh:], h0_ref[Bh:]),
        unroll=False)


def kernel(tokens, c0, h0, table, wi, bi, wh, bh):
    T, B = tokens.shape
    V, H = table.shape

    x_emb = jnp.take(table, tokens, axis=0)
    b = bi + bh
    wi16 = wi.astype(jnp.bfloat16)
    wh16 = wh.astype(jnp.bfloat16)

    cy_seq, hy_seq = pl.pallas_call(
        _lstm_seq_kernel,
        out_shape=(jax.ShapeDtypeStruct((T, B, H), jnp.float32),
                   jax.ShapeDtypeStruct((T, B, H), jnp.float32)),
        grid=(1,),
        in_specs=[
            pl.BlockSpec((T, B, H),   lambda bidx: (0, 0, 0)),
            pl.BlockSpec((H, 4 * H),  lambda bidx: (0, 0)),
            pl.BlockSpec((H, 4 * H),  lambda bidx: (0, 0)),
            pl.BlockSpec((1, 4 * H),  lambda bidx: (0, 0)),
            pl.BlockSpec((B, H),      lambda bidx: (0, 0)),
            pl.BlockSpec((B, H),      lambda bidx: (0, 0)),
        ],
        out_specs=(pl.BlockSpec((T, B, H), lambda bidx: (0, 0, 0)),
                   pl.BlockSpec((T, B, H), lambda bidx: (0, 0, 0))),
        compiler_params=pltpu.CompilerParams(
            dimension_semantics=("arbitrary",),
            vmem_limit_bytes=100 * 1024 * 1024,
        ),
    )(x_emb, wi16, wh16, b, c0, h0)

    return cy_seq, hy_seq


# speedup vs baseline: 1.4604x; 1.4604x over previous
import jax
import jax.numpy as jnp
from jax.experimental import pallas as pl
from jax.experimental.pallas import tpu as pltpu


def _lstm_seq_kernel(x_ref,
                     wi_ref,
                     wh_ref,
                     b_ref,
                     c0_ref,
                     h0_ref,
                     cy_ref,
                     hy_ref):
    T = x_ref.shape[0]
    B = c0_ref.shape[0]
    H = c0_ref.shape[1]
    Bh = B // 2

    def step(t, carry):
        c, h = carry
        gates = (jnp.dot(x_ref[t].astype(jnp.bfloat16), wi_ref[...],
                         preferred_element_type=jnp.float32)
                 + jnp.dot(h.astype(jnp.bfloat16), wh_ref[...],
                           preferred_element_type=jnp.float32)
                 + b_ref[...])
        ingate     = jax.nn.sigmoid(gates[:, 0 * H:1 * H])
        forgetgate = jax.nn.sigmoid(gates[:, 1 * H:2 * H])
        cellgate   = jnp.tanh(gates[:, 2 * H:3 * H])
        outgate    = jax.nn.sigmoid(gates[:, 3 * H:4 * H])
        cy = forgetgate * c + ingate * cellgate
        hy = outgate * jnp.tanh(cy)
        cy_ref[t] = cy
        hy_ref[t] = hy
        return (cy, hy)

    jax.lax.fori_loop(0, T, step, (c0_ref[...], h0_ref[...]),
                      unroll=4)


def kernel(tokens, c0, h0, table, wi, bi, wh, bh):
    T, B = tokens.shape
    V, H = table.shape

    x_emb = jnp.take(table, tokens, axis=0)
    b = bi + bh
    wi16 = wi.astype(jnp.bfloat16)
    wh16 = wh.astype(jnp.bfloat16)

    cy_seq, hy_seq = pl.pallas_call(
        _lstm_seq_kernel,
        out_shape=(jax.ShapeDtypeStruct((T, B, H), jnp.float32),
                   jax.ShapeDtypeStruct((T, B, H), jnp.float32)),
        grid=(1,),
        in_specs=[
            pl.BlockSpec((T, B, H),   lambda bidx: (0, 0, 0)),
            pl.BlockSpec((H, 4 * H),  lambda bidx: (0, 0)),
            pl.BlockSpec((H, 4 * H),  lambda bidx: (0, 0)),
            pl.BlockSpec((1, 4 * H),  lambda bidx: (0, 0)),
            pl.BlockSpec((B, H),      lambda bidx: (0, 0)),
            pl.BlockSpec((B, H),      lambda bidx: (0, 0)),
        ],
        out_specs=(pl.BlockSpec((T, B, H), lambda bidx: (0, 0, 0)),
                   pl.BlockSpec((T, B, H), lambda bidx: (0, 0, 0))),
        compiler_params=pltpu.CompilerParams(
            dimension_semantics=("arbitrary",),
            vmem_limit_bytes=100 * 1024 * 1024,
        ),
    )(x_emb, wi16, wh16, b, c0, h0)

    return cy_seq, hy_seq


# speedup vs baseline: 1.4827x; 1.0152x over previous
import jax
import jax.numpy as jnp
from jax.experimental import pallas as pl
from jax.experimental.pallas import tpu as pltpu


def _lstm_seq_kernel(x_ref,
                     wi_ref,
                     wh_ref,
                     b_ref,
                     c0_ref,
                     h0_ref,
                     cy_ref,
                     hy_ref):
    T = x_ref.shape[0]
    B = c0_ref.shape[0]
    H = c0_ref.shape[1]
    Bh = B // 2

    def step(t, carry):
        c, h = carry
        gates = (jnp.dot(x_ref[t].astype(jnp.bfloat16), wi_ref[...],
                         preferred_element_type=jnp.float32)
                 + jnp.dot(h.astype(jnp.bfloat16), wh_ref[...],
                           preferred_element_type=jnp.float32)
                 + b_ref[...])
        ingate     = jax.nn.sigmoid(gates[:, 0 * H:1 * H])
        forgetgate = jax.nn.sigmoid(gates[:, 1 * H:2 * H])
        cellgate   = jnp.tanh(gates[:, 2 * H:3 * H])
        outgate    = jax.nn.sigmoid(gates[:, 3 * H:4 * H])
        cy = forgetgate * c + ingate * cellgate
        hy = outgate * jnp.tanh(cy)
        cy_ref[t] = cy
        hy_ref[t] = hy
        return (cy, hy)

    jax.lax.fori_loop(0, T, step, (c0_ref[...], h0_ref[...]),
                      unroll=8)


def kernel(tokens, c0, h0, table, wi, bi, wh, bh):
    T, B = tokens.shape
    V, H = table.shape

    x_emb = jnp.take(table, tokens, axis=0)
    b = bi + bh
    wi16 = wi.astype(jnp.bfloat16)
    wh16 = wh.astype(jnp.bfloat16)

    cy_seq, hy_seq = pl.pallas_call(
        _lstm_seq_kernel,
        out_shape=(jax.ShapeDtypeStruct((T, B, H), jnp.float32),
                   jax.ShapeDtypeStruct((T, B, H), jnp.float32)),
        grid=(1,),
        in_specs=[
            pl.BlockSpec((T, B, H),   lambda bidx: (0, 0, 0)),
            pl.BlockSpec((H, 4 * H),  lambda bidx: (0, 0)),
            pl.BlockSpec((H, 4 * H),  lambda bidx: (0, 0)),
            pl.BlockSpec((1, 4 * H),  lambda bidx: (0, 0)),
            pl.BlockSpec((B, H),      lambda bidx: (0, 0)),
            pl.BlockSpec((B, H),      lambda bidx: (0, 0)),
        ],
        out_specs=(pl.BlockSpec((T, B, H), lambda bidx: (0, 0, 0)),
                   pl.BlockSpec((T, B, H), lambda bidx: (0, 0, 0))),
        compiler_params=pltpu.CompilerParams(
            dimension_semantics=("arbitrary",),
            vmem_limit_bytes=100 * 1024 * 1024,
        ),
    )(x_emb, wi16, wh16, b, c0, h0)

    return cy_seq, hy_seq


# speedup vs baseline: 1.6470x; 1.1108x over previous
import jax
import jax.numpy as jnp
from jax.experimental import pallas as pl
from jax.experimental.pallas import tpu as pltpu

_TIME_CHUNK = 16


def _lstm_seq_kernel(x_ref,
                     wi_ref,
                     wh_ref,
                     b_ref,
                     c0_ref,
                     h0_ref,
                     cy_ref,
                     hy_ref,
                     c_st, h_st):
    Tc = x_ref.shape[0]
    H = c0_ref.shape[1]

    @pl.when(pl.program_id(0) == 0)
    def _():
        c_st[...] = c0_ref[...]
        h_st[...] = h0_ref[...]

    def step(t, carry):
        c, h = carry
        gates = (jnp.dot(x_ref[t].astype(jnp.bfloat16), wi_ref[...],
                         preferred_element_type=jnp.float32)
                 + jnp.dot(h.astype(jnp.bfloat16), wh_ref[...],
                           preferred_element_type=jnp.float32)
                 + b_ref[...])
        ingate     = jax.nn.sigmoid(gates[:, 0 * H:1 * H])
        forgetgate = jax.nn.sigmoid(gates[:, 1 * H:2 * H])
        cellgate   = jnp.tanh(gates[:, 2 * H:3 * H])
        outgate    = jax.nn.sigmoid(gates[:, 3 * H:4 * H])
        cy = forgetgate * c + ingate * cellgate
        hy = outgate * jnp.tanh(cy)
        cy_ref[t] = cy
        hy_ref[t] = hy
        return (cy, hy)

    cy, hy = jax.lax.fori_loop(0, Tc, step, (c_st[...], h_st[...]),
                               unroll=8)
    c_st[...] = cy
    h_st[...] = hy


def kernel(tokens, c0, h0, table, wi, bi, wh, bh):
    T, B = tokens.shape
    V, H = table.shape
    Tc = _TIME_CHUNK if T % _TIME_CHUNK == 0 else T

    x_emb = jnp.take(table, tokens, axis=0)
    b = bi + bh
    wi16 = wi.astype(jnp.bfloat16)
    wh16 = wh.astype(jnp.bfloat16)

    cy_seq, hy_seq = pl.pallas_call(
        _lstm_seq_kernel,
        out_shape=(jax.ShapeDtypeStruct((T, B, H), jnp.float32),
                   jax.ShapeDtypeStruct((T, B, H), jnp.float32)),
        grid=(T // Tc,),
        in_specs=[
            pl.BlockSpec((Tc, B, H),  lambda i: (i, 0, 0)),
            pl.BlockSpec((H, 4 * H),  lambda i: (0, 0)),
            pl.BlockSpec((H, 4 * H),  lambda i: (0, 0)),
            pl.BlockSpec((1, 4 * H),  lambda i: (0, 0)),
            pl.BlockSpec((B, H),      lambda i: (0, 0)),
            pl.BlockSpec((B, H),      lambda i: (0, 0)),
        ],
        out_specs=(pl.BlockSpec((Tc, B, H), lambda i: (i, 0, 0)),
                   pl.BlockSpec((Tc, B, H), lambda i: (i, 0, 0))),
        scratch_shapes=[
            pltpu.VMEM((B, H), jnp.float32),
            pltpu.VMEM((B, H), jnp.float32),
        ],
        compiler_params=pltpu.CompilerParams(
            dimension_semantics=("arbitrary",),
            vmem_limit_bytes=100 * 1024 * 1024,
        ),
    )(x_emb, wi16, wh16, b, c0, h0)

    return cy_seq, hy_seq


# speedup vs baseline: 1.6506x; 1.0022x over previous
import jax
import jax.numpy as jnp
from jax.experimental import pallas as pl
from jax.experimental.pallas import tpu as pltpu

_TIME_CHUNK = 16


def _lstm_seq_kernel(x_ref,
                     wi_ref,
                     wh_ref,
                     b_ref,
                     c0_ref,
                     h0_ref,
                     cy_ref,
                     hy_ref,
                     c_st, h_st):
    Tc = x_ref.shape[0]
    H = c0_ref.shape[1]

    @pl.when(pl.program_id(0) == 0)
    def _():
        c_st[...] = c0_ref[...]
        h_st[...] = h0_ref[...]

    def step(t, carry):
        c, h = carry
        gates = (jnp.dot(x_ref[t].astype(jnp.bfloat16), wi_ref[...],
                         preferred_element_type=jnp.float32)
                 + jnp.dot(h.astype(jnp.bfloat16), wh_ref[...],
                           preferred_element_type=jnp.float32)
                 + b_ref[...])
        ingate     = jax.nn.sigmoid(gates[:, 0 * H:1 * H])
        forgetgate = jax.nn.sigmoid(gates[:, 1 * H:2 * H])
        cellgate   = jnp.tanh(gates[:, 2 * H:3 * H])
        outgate    = jax.nn.sigmoid(gates[:, 3 * H:4 * H])
        cy = forgetgate * c + ingate * cellgate
        hy = outgate * jnp.tanh(cy)
        cy_ref[t] = cy
        hy_ref[t] = hy
        return (cy, hy)

    cy, hy = jax.lax.fori_loop(0, Tc, step, (c_st[...], h_st[...]),
                               unroll=16)
    c_st[...] = cy
    h_st[...] = hy


def kernel(tokens, c0, h0, table, wi, bi, wh, bh):
    T, B = tokens.shape
    V, H = table.shape
    Tc = _TIME_CHUNK if T % _TIME_CHUNK == 0 else T

    x_emb = jnp.take(table, tokens, axis=0)
    b = bi + bh
    wi16 = wi.astype(jnp.bfloat16)
    wh16 = wh.astype(jnp.bfloat16)

    cy_seq, hy_seq = pl.pallas_call(
        _lstm_seq_kernel,
        out_shape=(jax.ShapeDtypeStruct((T, B, H), jnp.float32),
                   jax.ShapeDtypeStruct((T, B, H), jnp.float32)),
        grid=(T // Tc,),
        in_specs=[
            pl.BlockSpec((Tc, B, H),  lambda i: (i, 0, 0)),
            pl.BlockSpec((H, 4 * H),  lambda i: (0, 0)),
            pl.BlockSpec((H, 4 * H),  lambda i: (0, 0)),
            pl.BlockSpec((1, 4 * H),  lambda i: (0, 0)),
            pl.BlockSpec((B, H),      lambda i: (0, 0)),
            pl.BlockSpec((B, H),      lambda i: (0, 0)),
        ],
        out_specs=(pl.BlockSpec((Tc, B, H), lambda i: (i, 0, 0)),
                   pl.BlockSpec((Tc, B, H), lambda i: (i, 0, 0))),
        scratch_shapes=[
            pltpu.VMEM((B, H), jnp.float32),
            pltpu.VMEM((B, H), jnp.float32),
        ],
        compiler_params=pltpu.CompilerParams(
            dimension_semantics=("arbitrary",),
            vmem_limit_bytes=100 * 1024 * 1024,
        ),
    )(x_emb, wi16, wh16, b, c0, h0)

    return cy_seq, hy_seq


# speedup vs baseline: 1.6752x; 1.0149x over previous
import jax
import jax.numpy as jnp
from jax.experimental import pallas as pl
from jax.experimental.pallas import tpu as pltpu

_TIME_CHUNK = 8


def _lstm_seq_kernel(x_ref,
                     wi_ref,
                     wh_ref,
                     b_ref,
                     c0_ref,
                     h0_ref,
                     cy_ref,
                     hy_ref,
                     c_st, h_st):
    Tc = x_ref.shape[0]
    H = c0_ref.shape[1]

    @pl.when(pl.program_id(0) == 0)
    def _():
        c_st[...] = c0_ref[...]
        h_st[...] = h0_ref[...]

    def step(t, carry):
        c, h = carry
        gates = (jnp.dot(x_ref[t].astype(jnp.bfloat16), wi_ref[...],
                         preferred_element_type=jnp.float32)
                 + jnp.dot(h.astype(jnp.bfloat16), wh_ref[...],
                           preferred_element_type=jnp.float32)
                 + b_ref[...])
        ingate     = jax.nn.sigmoid(gates[:, 0 * H:1 * H])
        forgetgate = jax.nn.sigmoid(gates[:, 1 * H:2 * H])
        cellgate   = jnp.tanh(gates[:, 2 * H:3 * H])
        outgate    = jax.nn.sigmoid(gates[:, 3 * H:4 * H])
        cy = forgetgate * c + ingate * cellgate
        hy = outgate * jnp.tanh(cy)
        cy_ref[t] = cy
        hy_ref[t] = hy
        return (cy, hy)

    cy, hy = jax.lax.fori_loop(0, Tc, step, (c_st[...], h_st[...]),
                               unroll=8)
    c_st[...] = cy
    h_st[...] = hy


def kernel(tokens, c0, h0, table, wi, bi, wh, bh):
    T, B = tokens.shape
    V, H = table.shape
    Tc = _TIME_CHUNK if T % _TIME_CHUNK == 0 else T

    x_emb = jnp.take(table, tokens, axis=0)
    b = bi + bh
    wi16 = wi.astype(jnp.bfloat16)
    wh16 = wh.astype(jnp.bfloat16)

    cy_seq, hy_seq = pl.pallas_call(
        _lstm_seq_kernel,
        out_shape=(jax.ShapeDtypeStruct((T, B, H), jnp.float32),
                   jax.ShapeDtypeStruct((T, B, H), jnp.float32)),
        grid=(T // Tc,),
        in_specs=[
            pl.BlockSpec((Tc, B, H),  lambda i: (i, 0, 0)),
            pl.BlockSpec((H, 4 * H),  lambda i: (0, 0)),
            pl.BlockSpec((H, 4 * H),  lambda i: (0, 0)),
            pl.BlockSpec((1, 4 * H),  lambda i: (0, 0)),
            pl.BlockSpec((B, H),      lambda i: (0, 0)),
            pl.BlockSpec((B, H),      lambda i: (0, 0)),
        ],
        out_specs=(pl.BlockSpec((Tc, B, H), lambda i: (i, 0, 0)),
                   pl.BlockSpec((Tc, B, H), lambda i: (i, 0, 0))),
        scratch_shapes=[
            pltpu.VMEM((B, H), jnp.float32),
            pltpu.VMEM((B, H), jnp.float32),
        ],
        compiler_params=pltpu.CompilerParams(
            dimension_semantics=("arbitrary",),
            vmem_limit_bytes=100 * 1024 * 1024,
        ),
    )(x_emb, wi16, wh16, b, c0, h0)

    return cy_seq, hy_seq
